# baseline (speedup 1.0000x reference)
import functools

import jax
import jax.numpy as jnp
from jax import lax
from jax.experimental import pallas as pl
from jax.experimental.pallas import tpu as pltpu
from jax.experimental.pallas import tpu_sc as plsc

DIM = 64
N = 100000

NC = 2
NS = 16
NW = NC * NS
C = 128
CHUNKS = 25
B_PER_W = C * CHUNKS
B_PAD = B_PER_W * NW

_mesh = plsc.VectorSubcoreMesh(core_axis_name="c", subcore_axis_name="s")


@functools.partial(
    pl.kernel,
    mesh=_mesh,
    out_type=jax.ShapeDtypeStruct((B_PAD, DIM), jnp.float32),
    compiler_params=pltpu.CompilerParams(use_tc_tiling_on_sc=False),
    scratch_types=[
        pltpu.VMEM((B_PER_W,), jnp.int32),
        pltpu.VMEM((2, C, DIM), jnp.float32),
        pltpu.SemaphoreType.DMA,
        pltpu.SemaphoreType.DMA,
        pltpu.SemaphoreType.DMA,
        pltpu.SemaphoreType.DMA,
    ],
)
def _gather_kernel(idx_hbm, table_hbm, out_hbm, idx_v, rows_v,
                   gsem0, gsem1, ssem0, ssem1):
    wid = lax.axis_index("s") * NC + lax.axis_index("c")
    base = wid * B_PER_W
    gsems = (gsem0, gsem1)
    ssems = (ssem0, ssem1)

    pltpu.sync_copy(idx_hbm.at[pl.ds(base, B_PER_W)], idx_v)

    def idx_chunk(cj):
        return idx_v.at[pl.ds(cj * C, C)]

    pltpu.async_copy(table_hbm.at[idx_chunk(0)], rows_v.at[0], gsems[0])
    pltpu.async_copy(table_hbm.at[idx_chunk(1)], rows_v.at[1], gsems[1])

    @pl.loop(0, CHUNKS, step=2)
    def _step(j):
        for b in range(2):
            cj = j + b

            @pl.when(cj < CHUNKS)
            def _():
                pltpu.make_async_copy(
                    table_hbm.at[idx_chunk(cj)], rows_v.at[b], gsems[b]).wait()
                pltpu.async_copy(
                    rows_v.at[b], out_hbm.at[pl.ds(base + cj * C, C)],
                    ssems[b])
                nj = cj + 2

                @pl.when(nj < CHUNKS)
                def _():
                    pltpu.make_async_copy(
                        rows_v.at[b],
                        out_hbm.at[pl.ds(base + cj * C, C)], ssems[b]).wait()
                    pltpu.async_copy(
                        table_hbm.at[idx_chunk(nj)], rows_v.at[b], gsems[b])

    pltpu.make_async_copy(
        rows_v.at[0], out_hbm.at[pl.ds(base, C)], ssems[0]).wait()
    pltpu.make_async_copy(
        rows_v.at[1], out_hbm.at[pl.ds(base, C)], ssems[1]).wait()


def kernel(token_id, table):
    idx = jnp.concatenate(
        [token_id.astype(jnp.int32),
         jnp.zeros((B_PAD - N,), jnp.int32)])
    out = _gather_kernel(idx, table)
    return out[:N]


# speedup vs baseline: 1.0044x; 1.0044x over previous
import functools

import jax
import jax.numpy as jnp
from jax import lax
from jax.experimental import pallas as pl
from jax.experimental.pallas import tpu as pltpu
from jax.experimental.pallas import tpu_sc as plsc

DIM = 64
N = 100000

NC = 2
NS = 16
NW = NC * NS
C = 800
CHUNKS = 4
B_PER_W = C * CHUNKS
B_PAD = B_PER_W * NW

_mesh = plsc.VectorSubcoreMesh(core_axis_name="c", subcore_axis_name="s")


@functools.partial(
    pl.kernel,
    mesh=_mesh,
    out_type=jax.ShapeDtypeStruct((B_PAD, DIM), jnp.float32),
    compiler_params=pltpu.CompilerParams(use_tc_tiling_on_sc=False),
    scratch_types=[
        pltpu.VMEM((B_PER_W,), jnp.int32),
        pltpu.VMEM((2, C, DIM), jnp.float32),
        pltpu.SemaphoreType.DMA,
        pltpu.SemaphoreType.DMA,
        pltpu.SemaphoreType.DMA,
        pltpu.SemaphoreType.DMA,
    ],
)
def _gather_kernel(idx_hbm, table_hbm, out_hbm, idx_v, rows_v,
                   gsem0, gsem1, ssem0, ssem1):
    wid = lax.axis_index("s") * NC + lax.axis_index("c")
    base = wid * B_PER_W
    gsems = (gsem0, gsem1)
    ssems = (ssem0, ssem1)

    pltpu.sync_copy(idx_hbm.at[pl.ds(base, B_PER_W)], idx_v)

    def gather(cj, b):
        return pltpu.async_copy(
            table_hbm.at[idx_v.at[pl.ds(cj * C, C)]], rows_v.at[b], gsems[b])

    def store(cj, b):
        return pltpu.async_copy(
            rows_v.at[b], out_hbm.at[pl.ds(base + cj * C, C)], ssems[b])

    gathers = [gather(0, 0), gather(1, 1)]
    stores = [None, None]
    for cj in range(CHUNKS):
        b = cj % 2
        gathers[b].wait()
        stores[b] = store(cj, b)
        if cj + 2 < CHUNKS:
            stores[b].wait()
            gathers[b] = gather(cj + 2, b)
    stores[0].wait()
    stores[1].wait()


def kernel(token_id, table):
    idx = jnp.concatenate(
        [token_id.astype(jnp.int32),
         jnp.zeros((B_PAD - N,), jnp.int32)])
    out = _gather_kernel(idx, table)
    return out[:N]


# speedup vs baseline: 1.1575x; 1.1525x over previous
import functools

import jax
import jax.numpy as jnp
from jax import lax
from jax.experimental import pallas as pl
from jax.experimental.pallas import tpu as pltpu
from jax.experimental.pallas import tpu_sc as plsc

DIM = 64
N = 100000

NC = 2
NS = 16
NW = NC * NS
B_PER_W = 3128
C = 800
OFFS = (0, 800, 1600, B_PER_W - C)

_mesh = plsc.VectorSubcoreMesh(core_axis_name="c", subcore_axis_name="s")


@functools.partial(
    pl.kernel,
    mesh=_mesh,
    out_type=jax.ShapeDtypeStruct((N, DIM), jnp.float32),
    compiler_params=pltpu.CompilerParams(use_tc_tiling_on_sc=False),
    scratch_types=[
        pltpu.VMEM((B_PER_W,), jnp.int32),
        pltpu.VMEM((2, C, DIM), jnp.float32),
        pltpu.SemaphoreType.DMA,
        pltpu.SemaphoreType.DMA,
        pltpu.SemaphoreType.DMA,
        pltpu.SemaphoreType.DMA,
    ],
)
def _gather_kernel(idx_hbm, table_hbm, out_hbm, idx_v, rows_v,
                   gsem0, gsem1, ssem0, ssem1):
    wid = lax.axis_index("s") * NC + lax.axis_index("c")
    base = jnp.minimum(wid * B_PER_W, N - B_PER_W)
    gsems = (gsem0, gsem1)
    ssems = (ssem0, ssem1)

    pltpu.sync_copy(idx_hbm.at[pl.ds(base, B_PER_W)], idx_v)

    def gather(cj, b):
        return pltpu.async_copy(
            table_hbm.at[idx_v.at[pl.ds(OFFS[cj], C)]], rows_v.at[b],
            gsems[b])

    def store(cj, b):
        return pltpu.async_copy(
            rows_v.at[b], out_hbm.at[pl.ds(base + OFFS[cj], C)], ssems[b])

    gathers = [gather(0, 0), gather(1, 1)]
    stores = [None, None]
    for cj in range(len(OFFS)):
        b = cj % 2
        gathers[b].wait()
        stores[b] = store(cj, b)
        if cj + 2 < len(OFFS):
            stores[b].wait()
            gathers[b] = gather(cj + 2, b)
    stores[0].wait()
    stores[1].wait()


def kernel(token_id, table):
    return _gather_kernel(token_id.astype(jnp.int32), table)


# speedup vs baseline: 1.2508x; 1.0806x over previous
import functools

import jax
import jax.numpy as jnp
from jax import lax
from jax.experimental import pallas as pl
from jax.experimental.pallas import tpu as pltpu
from jax.experimental.pallas import tpu_sc as plsc

VOCAB = 1000000
DIM = 64
N = 100000

NC = 2
NS = 16
NW = NC * NS
B_PER_W = 3128
C = 400
OFFS = (0, 400, 800, 1200, 1600, 2000, 2400, B_PER_W - C)

_mesh = plsc.VectorSubcoreMesh(core_axis_name="c", subcore_axis_name="s")


@functools.partial(
    pl.kernel,
    mesh=_mesh,
    out_type=jax.ShapeDtypeStruct((N, DIM), jnp.float32),
    compiler_params=pltpu.CompilerParams(use_tc_tiling_on_sc=False),
    scratch_types=[
        pltpu.VMEM((B_PER_W,), jnp.int32),
        pltpu.VMEM((2, C, 2 * DIM), jnp.float32),
        pltpu.SemaphoreType.DMA,
        pltpu.SemaphoreType.DMA,
        pltpu.SemaphoreType.DMA,
        pltpu.SemaphoreType.DMA,
    ],
)
def _gather_kernel(idx_hbm, tablep_hbm, out_hbm, idx_v, rows_v,
                   gsem0, gsem1, ssem0, ssem1):
    wid = lax.axis_index("s") * NC + lax.axis_index("c")
    base = jnp.minimum(wid * B_PER_W, N - B_PER_W)
    gsems = (gsem0, gsem1)
    ssems = (ssem0, ssem1)

    pltpu.sync_copy(idx_hbm.at[pl.ds(base, B_PER_W)], idx_v)

    def gather(cj, b):
        return pltpu.async_copy(
            tablep_hbm.at[idx_v.at[pl.ds(OFFS[cj], C)]], rows_v.at[b],
            gsems[b])

    def store(cj, b):
        return pltpu.async_copy(
            rows_v.at[b, :, :DIM], out_hbm.at[pl.ds(base + OFFS[cj], C)],
            ssems[b])

    gathers = [gather(0, 0), gather(1, 1)]
    stores = [None, None]
    for cj in range(len(OFFS)):
        b = cj % 2
        gathers[b].wait()
        stores[b] = store(cj, b)
        if cj + 2 < len(OFFS):
            stores[b].wait()
            gathers[b] = gather(cj + 2, b)
    stores[0].wait()
    stores[1].wait()


def kernel(token_id, table):
    tablep = jnp.pad(table, ((0, 0), (0, DIM)))
    return _gather_kernel(token_id.astype(jnp.int32), tablep)
